# baseline (device time: 8524 ns/iter reference)
import jax
import jax.numpy as jnp
from jax import lax
from jax.experimental import pallas as pl
from jax.experimental.pallas import tpu as pltpu

K = 8


def _topk_rows(vals, k):
    neg = jnp.asarray(-jnp.inf, vals.dtype)
    out_cols = []
    for _ in range(k):
        mx = jnp.max(vals, axis=1, keepdims=True)
        out_cols.append(mx)
        vals = jnp.where(vals == mx, neg, vals)
    return jnp.concatenate(out_cols, axis=1)


def kernel(x):
    m, n = x.shape

    def body(x_ref, out_ref, local_buf, recv_buf, sems):
        my_x = lax.axis_index("x")
        my_y = lax.axis_index("y")
        nbr = (1 - my_x, my_y)

        barrier_sem = pltpu.get_barrier_semaphore()
        pl.semaphore_signal(
            barrier_sem, inc=1, device_id=nbr,
            device_id_type=pl.DeviceIdType.MESH,
        )

        local_buf[:, :] = _topk_rows(x_ref[:, :], K)

        pl.semaphore_wait(barrier_sem, 1)

        rdma = pltpu.make_async_remote_copy(
            src_ref=local_buf, dst_ref=recv_buf,
            send_sem=sems.at[0], recv_sem=sems.at[1],
            device_id=nbr, device_id_type=pl.DeviceIdType.MESH,
        )
        rdma.start()
        rdma.wait()

        out_ref[:, :] = _topk_rows(
            jnp.concatenate([local_buf[:, :], recv_buf[:, :]], axis=1), K
        )

    return pl.pallas_call(
        body,
        out_shape=jax.ShapeDtypeStruct((m, K), jnp.float32),
        in_specs=[pl.BlockSpec(memory_space=pltpu.VMEM)],
        out_specs=pl.BlockSpec(memory_space=pltpu.VMEM),
        scratch_shapes=[
            pltpu.VMEM((m, K), jnp.float32),
            pltpu.VMEM((m, K), jnp.float32),
            pltpu.SemaphoreType.DMA((2,)),
        ],
        compiler_params=pltpu.CompilerParams(collective_id=0),
    )(x)


# device time: 8370 ns/iter; 1.0184x vs baseline; 1.0184x over previous
import jax
import jax.numpy as jnp
from jax import lax
from jax.experimental import pallas as pl
from jax.experimental.pallas import tpu as pltpu

K = 8
N_CHUNKS = 2


def _topk_rows(vals, k):
    neg = jnp.asarray(-jnp.inf, vals.dtype)
    out_cols = []
    for _ in range(k):
        mx = jnp.max(vals, axis=1, keepdims=True)
        out_cols.append(mx)
        vals = jnp.where(vals == mx, neg, vals)
    return jnp.concatenate(out_cols, axis=1)


def kernel(x):
    m, n = x.shape
    mc = m // N_CHUNKS

    def body(x_ref, out_ref, local_buf, recv_buf, sems):
        my_x = lax.axis_index("x")
        my_y = lax.axis_index("y")
        nbr = (1 - my_x, my_y)

        barrier_sem = pltpu.get_barrier_semaphore()
        pl.semaphore_signal(
            barrier_sem, inc=1, device_id=nbr,
            device_id_type=pl.DeviceIdType.MESH,
        )

        rdmas = []
        for c in range(N_CHUNKS):
            rdmas.append(
                pltpu.make_async_remote_copy(
                    src_ref=local_buf.at[c],
                    dst_ref=recv_buf.at[c],
                    send_sem=sems.at[2 * c],
                    recv_sem=sems.at[2 * c + 1],
                    device_id=nbr,
                    device_id_type=pl.DeviceIdType.MESH,
                )
            )

        for c in range(N_CHUNKS):
            local_buf[c, :, :] = _topk_rows(x_ref[pl.ds(c * mc, mc), :], K)
            if c == 0:
                pl.semaphore_wait(barrier_sem, 1)
            rdmas[c].start()

        for c in range(N_CHUNKS):
            rdmas[c].wait_recv()
            out_ref[pl.ds(c * mc, mc), :] = _topk_rows(
                jnp.concatenate(
                    [local_buf[c, :, :], recv_buf[c, :, :]], axis=1
                ),
                K,
            )
        for c in range(N_CHUNKS):
            rdmas[c].wait_send()

    return pl.pallas_call(
        body,
        out_shape=jax.ShapeDtypeStruct((m, K), jnp.float32),
        in_specs=[pl.BlockSpec(memory_space=pltpu.VMEM)],
        out_specs=pl.BlockSpec(memory_space=pltpu.VMEM),
        scratch_shapes=[
            pltpu.VMEM((N_CHUNKS, mc, K), jnp.float32),
            pltpu.VMEM((N_CHUNKS, mc, K), jnp.float32),
            pltpu.SemaphoreType.DMA((2 * N_CHUNKS,)),
        ],
        compiler_params=pltpu.CompilerParams(collective_id=0),
    )(x)
